# baseline (speedup 1.0000x reference)
import functools

import jax
import jax.numpy as jnp
from jax import lax
from jax.experimental import pallas as pl
from jax.experimental.pallas import tpu as pltpu
from jax.experimental.pallas import tpu_sc as plsc

NUM_NEWS = 1000000
NUM_CATEGORIES = 1000
NEWS_DIM = 64
CATEGORY_DIM = 16
BATCH = 4096
SEQ_LEN = 50
TOTAL = BATCH * SEQ_LEN
JOINT_DIM = NEWS_DIM + CATEGORY_DIM
ROW_PAD = 128

NUM_CORES = 2
NUM_SUBCORES = 16
NW = NUM_CORES * NUM_SUBCORES
PER_W = TOTAL // NW
CHUNK = 128
N_CHUNK = PER_W // CHUNK
LANES = 16
NBUF = 2


def _sc_body(nidx2_hbm, nidx_hbm, cidx_hbm, news_hbm, cat_hbm, out_hbm,
             nidx2_v, nidx_v, cidx_v, cat_v, pair0_v, pair1_v,
             stage0_v, stage1_v, gsem0, gsem1, wsem0, wsem1):
    cid = lax.axis_index("c")
    sid = lax.axis_index("s")
    wid = sid * NUM_CORES + cid
    base_row = wid * PER_W
    pltpu.sync_copy(nidx2_hbm.at[pl.ds(base_row, PER_W)], nidx2_v)
    pltpu.sync_copy(nidx_hbm.at[pl.ds(base_row, PER_W)], nidx_v)
    pltpu.sync_copy(cidx_hbm.at[pl.ds(base_row, PER_W)], cidx_v)
    pltpu.sync_copy(cat_hbm, cat_v)

    pairs = (pair0_v, pair1_v)
    stages = (stage0_v, stage1_v)
    gsems = (gsem0, gsem1)
    wsems = (wsem0, wsem1)

    def gather_copy(c, b):
        idx_n = nidx2_v.at[pl.ds(c * CHUNK, CHUNK)]
        return pltpu.make_async_copy(news_hbm.at[idx_n], pairs[b], gsems[b])

    def write_copy(c, b):
        off = (base_row + c * CHUNK) * JOINT_DIM
        return pltpu.make_async_copy(stages[b],
                                     out_hbm.at[pl.ds(off, CHUNK * JOINT_DIM)],
                                     wsems[b])

    def merge(c, b):
        pv = pairs[b]
        sv = stages[b]
        iota = lax.iota(jnp.int32, LANES)

        def row_body(r, carry):
            rsplat = jnp.full((LANES,), c * CHUNK + r, jnp.int32)
            idv = plsc.load_gather(nidx_v, [rsplat])
            odd = idv >= HALF_NEWS
            cidv = plsc.load_gather(cidx_v, [rsplat])
            dbase = r * JOINT_DIM
            for k in range(NEWS_DIM // LANES):
                lo = pv[r, pl.ds(k * LANES, LANES)]
                hi = pv[r, pl.ds(NEWS_DIM + k * LANES, LANES)]
                sv[pl.ds(dbase + k * LANES, LANES)] = jnp.where(odd, hi, lo)
            cvals = plsc.load_gather(cat_v, [cidv * CATEGORY_DIM + iota])
            sv[pl.ds(dbase + NEWS_DIM, LANES)] = cvals
            return carry

        lax.fori_loop(0, CHUNK, row_body, 0)

    gather_copy(0, 0).start()

    def pair_body(g, carry):
        for b in range(NBUF):
            c = g * NBUF + b
            nc = c + 1
            @pl.when(nc < N_CHUNK)
            def _():
                gather_copy(nc, (b + 1) % NBUF).start()
            gather_copy(c, b).wait()
            @pl.when(c >= NBUF)
            def _():
                write_copy(c - NBUF, b).wait()
            merge(c, b)
            write_copy(c, b).start()
        return carry

    lax.fori_loop(0, N_CHUNK // NBUF, pair_body, 0)
    write_copy(N_CHUNK - 2, 0).wait()
    write_copy(N_CHUNK - 1, 1).wait()


HALF_NEWS = NUM_NEWS // 2
RELAY_BLK = 2000
RELAY_STEPS = HALF_NEWS // RELAY_BLK


def _relayout_body(a_ref, b_ref, o_ref):
    o_ref[:, 0:NEWS_DIM] = a_ref[...]
    o_ref[:, NEWS_DIM:ROW_PAD] = b_ref[...]


@jax.jit
def _relayout(news_table):
    return pl.pallas_call(
        _relayout_body,
        grid=(RELAY_STEPS,),
        in_specs=[
            pl.BlockSpec((RELAY_BLK, NEWS_DIM), lambda i: (i, 0)),
            pl.BlockSpec((RELAY_BLK, NEWS_DIM), lambda i: (i + RELAY_STEPS, 0)),
        ],
        out_specs=pl.BlockSpec((RELAY_BLK, ROW_PAD), lambda i: (i, 0)),
        out_shape=jax.ShapeDtypeStruct((HALF_NEWS, ROW_PAD), jnp.float32),
    )(news_table, news_table)


@jax.jit
def _joint_embed(news_idx2, news_idx, cat_idx, news128, cat_flat):
    mesh = plsc.VectorSubcoreMesh(core_axis_name="c", subcore_axis_name="s")
    f = functools.partial(
        pl.kernel,
        mesh=mesh,
        out_type=jax.ShapeDtypeStruct((TOTAL * JOINT_DIM,), jnp.float32),
        scratch_types=[
            pltpu.VMEM((PER_W,), jnp.int32),
            pltpu.VMEM((PER_W,), jnp.int32),
            pltpu.VMEM((PER_W,), jnp.int32),
            pltpu.VMEM((NUM_CATEGORIES * CATEGORY_DIM,), jnp.float32),
            pltpu.VMEM((CHUNK, ROW_PAD), jnp.float32),
            pltpu.VMEM((CHUNK, ROW_PAD), jnp.float32),
            pltpu.VMEM((CHUNK * JOINT_DIM,), jnp.float32),
            pltpu.VMEM((CHUNK * JOINT_DIM,), jnp.float32),
            pltpu.SemaphoreType.DMA,
            pltpu.SemaphoreType.DMA,
            pltpu.SemaphoreType.DMA,
            pltpu.SemaphoreType.DMA,
        ],
        compiler_params=pltpu.CompilerParams(needs_layout_passes=False),
    )(_sc_body)
    return f(news_idx2, news_idx, cat_idx, news128, cat_flat)


def kernel(news_ids, category_ids, news_table, category_table):
    news_idx = news_ids.reshape(TOTAL)
    news_idx2 = jnp.where(news_idx >= HALF_NEWS,
                          news_idx - HALF_NEWS, news_idx)
    cat_idx = category_ids.reshape(TOTAL)
    news128 = _relayout(news_table)
    cat_flat = category_table.reshape(NUM_CATEGORIES * CATEGORY_DIM)
    out = _joint_embed(news_idx2, news_idx, cat_idx, news128, cat_flat)
    return out.reshape(BATCH, SEQ_LEN, JOINT_DIM)


# speedup vs baseline: 1.2050x; 1.2050x over previous
import functools

import jax
import jax.numpy as jnp
from jax import lax
from jax.experimental import pallas as pl
from jax.experimental.pallas import tpu as pltpu
from jax.experimental.pallas import tpu_sc as plsc

NUM_NEWS = 1000000
NUM_CATEGORIES = 1000
NEWS_DIM = 64
CATEGORY_DIM = 16
BATCH = 4096
SEQ_LEN = 50
TOTAL = BATCH * SEQ_LEN
JOINT_DIM = NEWS_DIM + CATEGORY_DIM
ROW_PAD = 128
SEQ_PAD = 64

NUM_CORES = 2
NUM_SUBCORES = 16
NW = NUM_CORES * NUM_SUBCORES
BATCH_W = BATCH // NW
LANES = 16
NBUF = 2


def _sc_body(nidx2_hbm, nidx_hbm, cidx_hbm, news_hbm, cat_hbm, out_hbm,
             nidx2_v, nidx_v, cidx_v, cat_v, pair0_v, pair1_v,
             stage0_v, stage1_v, gsem0, gsem1, wsem0, wsem1):
    cid = lax.axis_index("c")
    sid = lax.axis_index("s")
    wid = sid * NUM_CORES + cid
    base = wid * BATCH_W * SEQ_PAD
    pltpu.sync_copy(nidx2_hbm.at[pl.ds(base, BATCH_W * SEQ_PAD)], nidx2_v)
    pltpu.sync_copy(nidx_hbm.at[pl.ds(base, BATCH_W * SEQ_PAD)], nidx_v)
    pltpu.sync_copy(cidx_hbm.at[pl.ds(base, BATCH_W * SEQ_PAD)], cidx_v)
    pltpu.sync_copy(cat_hbm, cat_v)

    pairs = (pair0_v, pair1_v)
    stages = (stage0_v, stage1_v)
    gsems = (gsem0, gsem1)
    wsems = (wsem0, wsem1)

    def gather_copy(b, s):
        idx_n = nidx2_v.at[pl.ds(b * SEQ_PAD, SEQ_LEN)]
        return pltpu.make_async_copy(news_hbm.at[idx_n], pairs[s], gsems[s])

    def write_copy(b, s):
        return pltpu.make_async_copy(stages[s],
                                     out_hbm.at[pl.ds(wid * BATCH_W + b, 1)],
                                     wsems[s])

    def merge(b, s):
        pv = pairs[s]
        sv = stages[s]
        iota = lax.iota(jnp.int32, LANES)

        def row_body(r, carry):
            rsplat = jnp.full((LANES,), b * SEQ_PAD + r, jnp.int32)
            idv = plsc.load_gather(nidx_v, [rsplat])
            odd = (idv & 1) != 0
            cidv = plsc.load_gather(cidx_v, [rsplat])
            for k in range(NEWS_DIM // LANES):
                lo = pv[r, pl.ds(k * LANES, LANES)]
                hi = pv[r, pl.ds(NEWS_DIM + k * LANES, LANES)]
                sv[0, r, pl.ds(k * LANES, LANES)] = jnp.where(odd, hi, lo)
            cvals = plsc.load_gather(cat_v, [cidv * CATEGORY_DIM + iota])
            sv[0, r, pl.ds(NEWS_DIM, LANES)] = cvals
            return carry

        lax.fori_loop(0, SEQ_LEN, row_body, 0)

    gather_copy(0, 0).start()

    def pair_body(g, carry):
        for s in range(NBUF):
            b = g * NBUF + s
            nb = b + 1
            @pl.when(nb < BATCH_W)
            def _():
                gather_copy(nb, (s + 1) % NBUF).start()
            gather_copy(b, s).wait()
            @pl.when(b >= NBUF)
            def _():
                write_copy(b - NBUF, s).wait()
            merge(b, s)
            write_copy(b, s).start()
        return carry

    lax.fori_loop(0, BATCH_W // NBUF, pair_body, 0)
    write_copy(BATCH_W - 2, 0).wait()
    write_copy(BATCH_W - 1, 1).wait()


@jax.jit
def _joint_embed(news_idx2, news_idx, cat_idx, news128, cat_flat):
    mesh = plsc.VectorSubcoreMesh(core_axis_name="c", subcore_axis_name="s")
    f = functools.partial(
        pl.kernel,
        mesh=mesh,
        out_type=jax.ShapeDtypeStruct((BATCH, SEQ_LEN, JOINT_DIM),
                                      jnp.float32),
        scratch_types=[
            pltpu.VMEM((BATCH_W * SEQ_PAD,), jnp.int32),
            pltpu.VMEM((BATCH_W * SEQ_PAD,), jnp.int32),
            pltpu.VMEM((BATCH_W * SEQ_PAD,), jnp.int32),
            pltpu.VMEM((NUM_CATEGORIES * CATEGORY_DIM,), jnp.float32),
            pltpu.VMEM((SEQ_LEN, ROW_PAD), jnp.float32),
            pltpu.VMEM((SEQ_LEN, ROW_PAD), jnp.float32),
            pltpu.VMEM((1, SEQ_LEN, JOINT_DIM), jnp.float32),
            pltpu.VMEM((1, SEQ_LEN, JOINT_DIM), jnp.float32),
            pltpu.SemaphoreType.DMA,
            pltpu.SemaphoreType.DMA,
            pltpu.SemaphoreType.DMA,
            pltpu.SemaphoreType.DMA,
        ],
        compiler_params=pltpu.CompilerParams(needs_layout_passes=False),
    )(_sc_body)
    return f(news_idx2, news_idx, cat_idx, news128, cat_flat)


def kernel(news_ids, category_ids, news_table, category_table):
    pad = ((0, 0), (0, SEQ_PAD - SEQ_LEN))
    news_idx = jnp.pad(news_ids, pad).reshape(BATCH * SEQ_PAD)
    news_idx2 = news_idx >> 1
    cat_idx = jnp.pad(category_ids, pad).reshape(BATCH * SEQ_PAD)
    news128 = news_table.reshape(NUM_NEWS // 2, ROW_PAD)
    cat_flat = category_table.reshape(NUM_CATEGORIES * CATEGORY_DIM)
    return _joint_embed(news_idx2, news_idx, cat_idx, news128, cat_flat)


# speedup vs baseline: 1.2586x; 1.0445x over previous
import functools

import jax
import jax.numpy as jnp
from jax import lax
from jax.experimental import pallas as pl
from jax.experimental.pallas import tpu as pltpu
from jax.experimental.pallas import tpu_sc as plsc

NUM_NEWS = 1000000
NUM_CATEGORIES = 1000
NEWS_DIM = 64
CATEGORY_DIM = 16
BATCH = 4096
SEQ_LEN = 50
TOTAL = BATCH * SEQ_LEN
JOINT_DIM = NEWS_DIM + CATEGORY_DIM
ROW_PAD = 128
SEQ_PAD = 64

NUM_CORES = 2
NUM_SUBCORES = 16
NW = NUM_CORES * NUM_SUBCORES
BATCH_W = BATCH // NW
LANES = 16
NBUF = 4
LEAD = 3


def _sc_body(nidx2_hbm, nidx_hbm, cidx_hbm, news_hbm, cat_hbm, out_hbm,
             nidx2_v, nidx_v, cidx_v, cat_v,
             pair0_v, pair1_v, pair2_v, pair3_v,
             stage0_v, stage1_v, stage2_v, stage3_v,
             gsem0, gsem1, gsem2, gsem3, wsem0, wsem1, wsem2, wsem3):
    cid = lax.axis_index("c")
    sid = lax.axis_index("s")
    wid = sid * NUM_CORES + cid
    base = wid * BATCH_W * SEQ_PAD
    pltpu.sync_copy(nidx2_hbm.at[pl.ds(base, BATCH_W * SEQ_PAD)], nidx2_v)
    pltpu.sync_copy(nidx_hbm.at[pl.ds(base, BATCH_W * SEQ_PAD)], nidx_v)
    pltpu.sync_copy(cidx_hbm.at[pl.ds(base, BATCH_W * SEQ_PAD)], cidx_v)
    pltpu.sync_copy(cat_hbm, cat_v)

    pairs = (pair0_v, pair1_v, pair2_v, pair3_v)
    stages = (stage0_v, stage1_v, stage2_v, stage3_v)
    gsems = (gsem0, gsem1, gsem2, gsem3)
    wsems = (wsem0, wsem1, wsem2, wsem3)

    def gather_copy(b, s):
        idx_n = nidx2_v.at[pl.ds(b * SEQ_PAD, SEQ_LEN)]
        return pltpu.make_async_copy(news_hbm.at[idx_n], pairs[s], gsems[s])

    def write_copy(b, s):
        return pltpu.make_async_copy(stages[s],
                                     out_hbm.at[pl.ds(wid * BATCH_W + b, 1)],
                                     wsems[s])

    def merge(b, s):
        pv = pairs[s]
        sv = stages[s]
        iota = lax.iota(jnp.int32, LANES)

        def row_body(r, carry):
            rsplat = jnp.full((LANES,), b * SEQ_PAD + r, jnp.int32)
            idv = plsc.load_gather(nidx_v, [rsplat])
            odd = (idv & 1) != 0
            cidv = plsc.load_gather(cidx_v, [rsplat])
            for k in range(NEWS_DIM // LANES):
                lo = pv[r, pl.ds(k * LANES, LANES)]
                hi = pv[r, pl.ds(NEWS_DIM + k * LANES, LANES)]
                sv[0, r, pl.ds(k * LANES, LANES)] = jnp.where(odd, hi, lo)
            cvals = plsc.load_gather(cat_v, [cidv * CATEGORY_DIM + iota])
            sv[0, r, pl.ds(NEWS_DIM, LANES)] = cvals
            return carry

        lax.fori_loop(0, SEQ_LEN, row_body, 0)

    for p in range(LEAD):
        gather_copy(p, p).start()

    def pair_body(g, carry):
        for s in range(NBUF):
            b = g * NBUF + s
            nb = b + LEAD
            @pl.when(nb < BATCH_W)
            def _():
                gather_copy(nb, (s + LEAD) % NBUF).start()
            gather_copy(b, s).wait()
            @pl.when(b >= NBUF)
            def _():
                write_copy(b - NBUF, s).wait()
            merge(b, s)
            write_copy(b, s).start()
        return carry

    lax.fori_loop(0, BATCH_W // NBUF, pair_body, 0)
    for p in range(NBUF):
        b = BATCH_W - NBUF + p
        write_copy(b, b % NBUF).wait()


@jax.jit
def _joint_embed(news_idx2, news_idx, cat_idx, news128, cat_flat):
    mesh = plsc.VectorSubcoreMesh(core_axis_name="c", subcore_axis_name="s")
    f = functools.partial(
        pl.kernel,
        mesh=mesh,
        out_type=jax.ShapeDtypeStruct((BATCH, SEQ_LEN, JOINT_DIM),
                                      jnp.float32),
        scratch_types=[
            pltpu.VMEM((BATCH_W * SEQ_PAD,), jnp.int32),
            pltpu.VMEM((BATCH_W * SEQ_PAD,), jnp.int32),
            pltpu.VMEM((BATCH_W * SEQ_PAD,), jnp.int32),
            pltpu.VMEM((NUM_CATEGORIES * CATEGORY_DIM,), jnp.float32),
            pltpu.VMEM((SEQ_LEN, ROW_PAD), jnp.float32),
            pltpu.VMEM((SEQ_LEN, ROW_PAD), jnp.float32),
            pltpu.VMEM((SEQ_LEN, ROW_PAD), jnp.float32),
            pltpu.VMEM((SEQ_LEN, ROW_PAD), jnp.float32),
            pltpu.VMEM((1, SEQ_LEN, JOINT_DIM), jnp.float32),
            pltpu.VMEM((1, SEQ_LEN, JOINT_DIM), jnp.float32),
            pltpu.VMEM((1, SEQ_LEN, JOINT_DIM), jnp.float32),
            pltpu.VMEM((1, SEQ_LEN, JOINT_DIM), jnp.float32),
            pltpu.SemaphoreType.DMA,
            pltpu.SemaphoreType.DMA,
            pltpu.SemaphoreType.DMA,
            pltpu.SemaphoreType.DMA,
            pltpu.SemaphoreType.DMA,
            pltpu.SemaphoreType.DMA,
            pltpu.SemaphoreType.DMA,
            pltpu.SemaphoreType.DMA,
        ],
        compiler_params=pltpu.CompilerParams(needs_layout_passes=False),
    )(_sc_body)
    return f(news_idx2, news_idx, cat_idx, news128, cat_flat)


def kernel(news_ids, category_ids, news_table, category_table):
    pad = ((0, 0), (0, SEQ_PAD - SEQ_LEN))
    news_idx = jnp.pad(news_ids, pad).reshape(BATCH * SEQ_PAD)
    news_idx2 = news_idx >> 1
    cat_idx = jnp.pad(category_ids, pad).reshape(BATCH * SEQ_PAD)
    news128 = news_table.reshape(NUM_NEWS // 2, ROW_PAD)
    cat_flat = category_table.reshape(NUM_CATEGORIES * CATEGORY_DIM)
    return _joint_embed(news_idx2, news_idx, cat_idx, news128, cat_flat)
